# Initial kernel scaffold; baseline (speedup 1.0000x reference)
import jax
import jax.numpy as jnp
from jax.experimental import pallas as pl


def kernel(cls_logits, anchors, roi_proposals, gt_bboxes, width, height):
    raise NotImplementedError("write your pallas kernel here")


# speedup vs baseline: 3.2670x; 3.2670x over previous
import jax
import jax.numpy as jnp
from jax.experimental import pallas as pl
from jax.experimental.pallas import tpu as pltpu

N = 20000
G = 64
C = 128
R = 160
NPAD = R * C


def _rpn_kernel(ab_ref, pb_ref, lg_ref, gtp_ref, out_ref):
    a0 = ab_ref[0]
    a1 = ab_ref[1]
    a2 = ab_ref[2]
    a3 = ab_ref[3]
    w = gtp_ref[G, 0]
    h = gtp_ref[G, 1]
    area_a = (a2 - a0) * (a3 - a1)
    inside = (a0 >= 0.0) & (a1 >= 0.0) & (a2 <= w) & (a3 <= h)
    row_i = jax.lax.broadcasted_iota(jnp.int32, (R, C), 0)
    col_i = jax.lax.broadcasted_iota(jnp.int32, (R, C), 1)
    idx = row_i * C + col_i

    def body(g, carry):
        biou, bg0, bg1, bg2, bg3, bestmask = carry
        g0 = gtp_ref[g, 0]

        g1 = gtp_ref[g, 1]
        g2 = gtp_ref[g, 2]
        g3 = gtp_ref[g, 3]
        area_g = (g2 - g0) * (g3 - g1)
        ww = jnp.maximum(jnp.minimum(a2, g2) - jnp.maximum(a0, g0), 0.0)
        hh = jnp.maximum(jnp.minimum(a3, g3) - jnp.maximum(a1, g1), 0.0)
        inter = ww * hh
        iou = inter / (area_a + area_g - inter + 1e-9)
        upd = iou > biou
        biou = jnp.where(upd, iou, biou)
        bg0 = jnp.where(upd, g0, bg0)
        bg1 = jnp.where(upd, g1, bg1)
        bg2 = jnp.where(upd, g2, bg2)
        bg3 = jnp.where(upd, g3, bg3)
        iou_in = jnp.where(inside, iou, -1.0)
        cm = jnp.max(iou_in)
        fidx = jnp.min(jnp.where(iou_in == cm, idx, NPAD))
        bestmask = bestmask | (idx == fidx)
        return biou, bg0, bg1, bg2, bg3, bestmask

    neg_inf = jnp.full((R, C), -jnp.inf, jnp.float32)
    zeros = jnp.zeros((R, C), jnp.float32)
    false_m = jnp.zeros((R, C), jnp.bool_)
    carry = (neg_inf, zeros, zeros, zeros, zeros, false_m)
    for g in range(G):
        carry = body(g, carry)
    biou, bg0, bg1, bg2, bg3, bestmask = carry

    pos = ((biou >= 0.7) & inside) | bestmask
    neg = (biou < 0.3) & (~pos) & inside
    posf = pos.astype(jnp.float32)
    negf = neg.astype(jnp.float32)

    iu_r = jax.lax.broadcasted_iota(jnp.int32, (C, C), 0)
    iu_c = jax.lax.broadcasted_iota(jnp.int32, (C, C), 1)
    upper = (iu_r <= iu_c).astype(jnp.float32)
    il_r = jax.lax.broadcasted_iota(jnp.int32, (R, R), 0)
    il_c = jax.lax.broadcasted_iota(jnp.int32, (R, R), 1)
    lower = (il_c < il_r).astype(jnp.float32)

    rowc_p = jax.lax.dot_general(posf, upper, (((1,), (0,)), ((), ())),
                                 preferred_element_type=jnp.float32)
    tot_p = rowc_p[:, C - 1:C]
    off_p = jax.lax.dot_general(lower, tot_p, (((1,), (0,)), ((), ())),
                                preferred_element_type=jnp.float32)
    cpos = rowc_p + off_p
    total_pos = off_p[R - 1, 0] + tot_p[R - 1, 0]

    rowc_n = jax.lax.dot_general(negf, upper, (((1,), (0,)), ((), ())),
                                 preferred_element_type=jnp.float32)
    tot_n = rowc_n[:, C - 1:C]
    off_n = jax.lax.dot_general(lower, tot_n, (((1,), (0,)), ((), ())),
                                preferred_element_type=jnp.float32)
    cneg = rowc_n + off_n
    total_neg = off_n[R - 1, 0] + tot_n[R - 1, 0]

    n_pos = jnp.minimum(total_pos, 128.0)
    pos_sel = pos & (cpos <= 128.0)
    thresh = 256.0 - n_pos
    n_neg = jnp.minimum(total_neg, thresh)
    neg_sel = neg & (cneg <= thresh)
    n_samp = n_pos + n_neg

    l0 = lg_ref[0]
    l1 = lg_ref[1]
    m = jnp.maximum(l0, l1)
    lse = m + jnp.log(jnp.exp(l0 - m) + jnp.exp(l1 - m))
    cls_sum = jnp.sum(jnp.where(pos_sel, l1 - lse, 0.0) +
                      jnp.where(neg_sel, l0 - lse, 0.0))
    cls_loss = -cls_sum / n_samp

    aw = a2 - a0
    ah = a3 - a1
    ax = a0 + 0.5 * aw
    ay = a1 + 0.5 * ah
    p0 = pb_ref[0]
    p1 = pb_ref[1]
    p2 = pb_ref[2]
    p3 = pb_ref[3]
    pw = p2 - p0
    ph = p3 - p1
    px = p0 + 0.5 * pw
    py = p1 + 0.5 * ph
    gw = bg2 - bg0
    gh = bg3 - bg1
    gx = bg0 + 0.5 * gw
    gy = bg1 + 0.5 * gh
    dx = jnp.abs((px - ax) / aw - (gx - ax) / aw)
    dy = jnp.abs((py - ay) / ah - (gy - ay) / ah)
    dw = jnp.abs(jnp.log(pw / aw) - jnp.log(gw / aw))
    dh = jnp.abs(jnp.log(ph / ah) - jnp.log(gh / ah))
    esum = (jnp.where(dx < 1.0, 0.5 * dx * dx, dx - 0.5) +
            jnp.where(dy < 1.0, 0.5 * dy * dy, dy - 0.5) +
            jnp.where(dw < 1.0, 0.5 * dw * dw, dw - 0.5) +
            jnp.where(dh < 1.0, 0.5 * dh * dh, dh - 0.5))
    num = jnp.sum(jnp.where(pos_sel, esum, 0.0))
    bbox_loss = jnp.where(n_pos > 0.0, num / (4.0 * jnp.maximum(n_pos, 1.0)),
                          0.0)
    out_ref[0, 0] = cls_loss + bbox_loss


def kernel(cls_logits, anchors, roi_proposals, gt_bboxes, width, height):
    pad_box = jnp.array([-2.0, -2.0, -1.0, -1.0], jnp.float32)
    pad_a = jnp.broadcast_to(pad_box, (NPAD - N, 4))
    ab = jnp.concatenate([anchors, pad_a], 0).T.reshape(4, R, C)
    pb = jnp.concatenate([roi_proposals, pad_a], 0).T.reshape(4, R, C)
    lg = jnp.concatenate(
        [cls_logits[0], jnp.zeros((NPAD - N, 2), jnp.float32)], 0
    ).T.reshape(2, R, C)
    wh = jnp.stack([jnp.float32(width), jnp.float32(height),
                    jnp.float32(0), jnp.float32(0)])
    gtp = jnp.concatenate([gt_bboxes, wh[None, :]], 0)

    res = pl.pallas_call(
        _rpn_kernel,
        out_shape=jax.ShapeDtypeStruct((1, 1), jnp.float32),
        in_specs=[
            pl.BlockSpec(memory_space=pltpu.VMEM),
            pl.BlockSpec(memory_space=pltpu.VMEM),
            pl.BlockSpec(memory_space=pltpu.VMEM),
            pl.BlockSpec(memory_space=pltpu.SMEM),
        ],
        out_specs=pl.BlockSpec(memory_space=pltpu.SMEM),
    )(ab, pb, lg, gtp)
    return res[0, 0]


# speedup vs baseline: 5.5902x; 1.7111x over previous
import jax
import jax.numpy as jnp
from jax.experimental import pallas as pl
from jax.experimental.pallas import tpu as pltpu

N = 20000
G = 64
C = 128
R = 160
NPAD = R * C
CH = 16
NCH = R // CH


def _rpn_kernel(x_ref, gtp_ref, out_ref, siou_ref, sbiou_ref, sbg_ref):
    w = gtp_ref[G, 0]
    h = gtp_ref[G, 1]

    for ch in range(NCH):
        sl = pl.ds(ch * CH, CH)
        a0 = x_ref[0, sl, :]
        a1 = x_ref[1, sl, :]
        a2 = x_ref[2, sl, :]
        a3 = x_ref[3, sl, :]
        area_a = (a2 - a0) * (a3 - a1)
        inside_c = (a0 >= 0.0) & (a1 >= 0.0) & (a2 <= w) & (a3 <= h)
        biou = jnp.full((CH, C), -jnp.inf, jnp.float32)
        bg0 = jnp.zeros((CH, C), jnp.float32)
        bg1 = bg0
        bg2 = bg0
        bg3 = bg0
        for g in range(G):
            g0 = gtp_ref[g, 0]
            g1 = gtp_ref[g, 1]
            g2 = gtp_ref[g, 2]
            g3 = gtp_ref[g, 3]
            area_g = (g2 - g0) * (g3 - g1)
            ww = jnp.maximum(jnp.minimum(a2, g2) - jnp.maximum(a0, g0), 0.0)
            hh = jnp.maximum(jnp.minimum(a3, g3) - jnp.maximum(a1, g1), 0.0)
            inter = ww * hh
            iou = inter / (area_a + area_g - inter + 1e-9)
            upd = iou > biou
            biou = jnp.where(upd, iou, biou)
            bg0 = jnp.where(upd, g0, bg0)
            bg1 = jnp.where(upd, g1, bg1)
            bg2 = jnp.where(upd, g2, bg2)
            bg3 = jnp.where(upd, g3, bg3)
            siou_ref[g, sl, :] = jnp.where(inside_c, iou, -1.0)
        sbiou_ref[sl, :] = biou
        sbg_ref[0, sl, :] = bg0
        sbg_ref[1, sl, :] = bg1
        sbg_ref[2, sl, :] = bg2
        sbg_ref[3, sl, :] = bg3

    row_i = jax.lax.broadcasted_iota(jnp.int32, (R, C), 0)
    cm_rows = []
    r1_rows = []
    for g in range(G):
        sg = siou_ref[g]
        cmb = jnp.max(sg, axis=0, keepdims=True)
        r1 = jnp.min(jnp.where(sg == cmb, row_i, R), axis=0,
                     keepdims=True)
        cm_rows.append(cmb)
        r1_rows.append(r1)
    cm = jnp.concatenate(cm_rows, axis=0)
    r1 = jnp.concatenate(r1_rows, axis=0)
    colmax = jnp.max(cm, axis=1, keepdims=True)
    lane_g = jax.lax.broadcasted_iota(jnp.int32, (G, C), 1)
    cand = jnp.where(cm == colmax, r1 * C + lane_g, NPAD)
    fidx = jnp.min(cand, axis=1, keepdims=True)
    fr = fidx // C
    fc = fidx - fr * C
    pmat = (fr == jax.lax.broadcasted_iota(jnp.int32, (G, R), 1)
            ).astype(jnp.float32)
    qmat = (fc == jax.lax.broadcasted_iota(jnp.int32, (G, C), 1)
            ).astype(jnp.float32)
    bestcnt = jax.lax.dot_general(pmat, qmat, (((0,), (0,)), ((), ())),
                                  preferred_element_type=jnp.float32)
    bestmask = bestcnt > 0.5

    a0 = x_ref[0]
    a1 = x_ref[1]
    a2 = x_ref[2]
    a3 = x_ref[3]
    inside = (a0 >= 0.0) & (a1 >= 0.0) & (a2 <= w) & (a3 <= h)
    biou = sbiou_ref[:, :]
    pos = ((biou >= 0.7) & inside) | bestmask
    neg = (biou < 0.3) & (~pos) & inside
    posf = pos.astype(jnp.float32)
    negf = neg.astype(jnp.float32)

    iu_r = jax.lax.broadcasted_iota(jnp.int32, (C, C), 0)
    iu_c = jax.lax.broadcasted_iota(jnp.int32, (C, C), 1)
    upper = (iu_r <= iu_c).astype(jnp.float32)
    il_r = jax.lax.broadcasted_iota(jnp.int32, (R, R), 0)
    il_c = jax.lax.broadcasted_iota(jnp.int32, (R, R), 1)
    lower = (il_c < il_r).astype(jnp.float32)

    rowc_p = jax.lax.dot_general(posf, upper, (((1,), (0,)), ((), ())),
                                 preferred_element_type=jnp.float32)
    tot_p = rowc_p[:, C - 1:C]
    off_p = jax.lax.dot_general(lower, tot_p, (((1,), (0,)), ((), ())),
                                preferred_element_type=jnp.float32)
    cpos = rowc_p + off_p
    total_pos = off_p[R - 1, 0] + tot_p[R - 1, 0]

    rowc_n = jax.lax.dot_general(negf, upper, (((1,), (0,)), ((), ())),
                                 preferred_element_type=jnp.float32)
    tot_n = rowc_n[:, C - 1:C]
    off_n = jax.lax.dot_general(lower, tot_n, (((1,), (0,)), ((), ())),
                                preferred_element_type=jnp.float32)
    cneg = rowc_n + off_n
    total_neg = off_n[R - 1, 0] + tot_n[R - 1, 0]

    n_pos = jnp.minimum(total_pos, 128.0)
    pos_sel = pos & (cpos <= 128.0)
    thresh = 256.0 - n_pos
    n_neg = jnp.minimum(total_neg, thresh)
    neg_sel = neg & (cneg <= thresh)
    n_samp = n_pos + n_neg

    l0 = x_ref[8]
    l1 = x_ref[9]
    m = jnp.maximum(l0, l1)
    lse = m + jnp.log(jnp.exp(l0 - m) + jnp.exp(l1 - m))
    cls_sum = jnp.sum(jnp.where(pos_sel, l1 - lse, 0.0) +
                      jnp.where(neg_sel, l0 - lse, 0.0))
    cls_loss = -cls_sum / n_samp

    aw = a2 - a0
    ah = a3 - a1
    ax = a0 + 0.5 * aw
    ay = a1 + 0.5 * ah
    p0 = x_ref[4]
    p1 = x_ref[5]
    p2 = x_ref[6]
    p3 = x_ref[7]
    pw = p2 - p0
    ph = p3 - p1
    px = p0 + 0.5 * pw
    py = p1 + 0.5 * ph
    bg0 = sbg_ref[0]
    bg1 = sbg_ref[1]
    bg2 = sbg_ref[2]
    bg3 = sbg_ref[3]
    gw = bg2 - bg0
    gh = bg3 - bg1
    gx = bg0 + 0.5 * gw
    gy = bg1 + 0.5 * gh
    dx = jnp.abs((px - ax) / aw - (gx - ax) / aw)
    dy = jnp.abs((py - ay) / ah - (gy - ay) / ah)
    dw = jnp.abs(jnp.log(pw / aw) - jnp.log(gw / aw))
    dh = jnp.abs(jnp.log(ph / ah) - jnp.log(gh / ah))
    esum = (jnp.where(dx < 1.0, 0.5 * dx * dx, dx - 0.5) +
            jnp.where(dy < 1.0, 0.5 * dy * dy, dy - 0.5) +
            jnp.where(dw < 1.0, 0.5 * dw * dw, dw - 0.5) +
            jnp.where(dh < 1.0, 0.5 * dh * dh, dh - 0.5))
    num = jnp.sum(jnp.where(pos_sel, esum, 0.0))
    bbox_loss = jnp.where(n_pos > 0.0, num / (4.0 * jnp.maximum(n_pos, 1.0)),
                          0.0)
    out_ref[0, 0] = cls_loss + bbox_loss


def kernel(cls_logits, anchors, roi_proposals, gt_bboxes, width, height):
    packed = jnp.concatenate([anchors, roi_proposals, cls_logits[0]], axis=1)
    pad_row = jnp.array([-2.0, -2.0, -1.0, -1.0,
                         -2.0, -2.0, -1.0, -1.0, 0.0, 0.0], jnp.float32)
    pad = jnp.broadcast_to(pad_row, (NPAD - N, 10))
    x = jnp.concatenate([packed, pad], 0).T.reshape(10, R, C)
    wh = jnp.stack([jnp.float32(width), jnp.float32(height),
                    jnp.float32(0), jnp.float32(0)])
    gtp = jnp.concatenate([gt_bboxes, wh[None, :]], 0)

    res = pl.pallas_call(
        _rpn_kernel,
        out_shape=jax.ShapeDtypeStruct((1, 1), jnp.float32),
        in_specs=[
            pl.BlockSpec(memory_space=pltpu.VMEM),
            pl.BlockSpec(memory_space=pltpu.SMEM),
        ],
        out_specs=pl.BlockSpec(memory_space=pltpu.SMEM),
        scratch_shapes=[
            pltpu.VMEM((G, R, C), jnp.float32),
            pltpu.VMEM((R, C), jnp.float32),
            pltpu.VMEM((4, R, C), jnp.float32),
        ],
    )(x, gtp)
    return res[0, 0]
